# baseline (speedup 1.0000x reference)
import numpy as np
import jax
import jax.numpy as jnp
from jax.experimental import pallas as pl
from jax.experimental.pallas import tpu as pltpu

_TWO_PI = float(2.0 * np.pi)
_HALF_PI = float(0.5 * np.pi)
_DN_T = (((1,), (1,)), ((), ()))


def _upsample2_matrix(L):
    Lout = 2 * L
    Wm = np.zeros((Lout, L), dtype=np.float32)
    if L == 1:
        Wm[:, 0] = 1.0
        return Wm
    j = np.arange(Lout, dtype=np.float32)
    pos = j * (L - 1) / (Lout - 1)
    i0 = np.clip(np.floor(pos).astype(np.int64), 0, L - 2)
    frac = (pos - i0).astype(np.float32)
    Wm[np.arange(Lout), i0] = 1.0 - frac
    Wm[np.arange(Lout), i0 + 1] = frac
    return Wm


def _traj_matrix(L, doublings):
    Wm = np.eye(L, dtype=np.float32)
    cur = L
    for _ in range(doublings):
        Wm = _upsample2_matrix(cur) @ Wm
        cur *= 2
    return Wm


def _fused_ndft_kernel(G, H, M, W,
                       xr_ref, xi_ref, aux_ref,
                       out_ref, psum_ref,
                       wb_s, eyc_s, eys_s, uc_s):
    f32 = jnp.float32
    bf16 = jnp.bfloat16
    i = pl.program_id(0)
    n = pl.num_programs(0)

    @pl.when(i == 0)
    def _build_tables():
        aux = aux_ref[...]
        ax = aux[0:M, 0:1]
        xp = (jax.lax.broadcasted_iota(jnp.int32, (M, W), 1)
              .astype(f32) - float(W // 2))
        ph = ax * xp
        cph = jnp.cos(ph)
        sph = jnp.sin(ph)
        wb_s[0:M, 0:W] = cph.astype(bf16)
        wb_s[M:2 * M, 0:W] = (-sph).astype(bf16)
        wb_s[0:M, W:2 * W] = sph.astype(bf16)
        wb_s[M:2 * M, W:2 * W] = cph.astype(bf16)
        ay = aux[M:M + 1, :]
        yp = (jax.lax.broadcasted_iota(jnp.int32, (H, M), 0)
              .astype(f32) - float(H // 2))
        ph_y = yp * ay
        eyc_s[...] = jnp.cos(ph_y).astype(bf16)
        eys_s[...] = jnp.sin(ph_y).astype(bf16)

    xr = xr_ref[...].astype(bf16)
    xi = xi_ref[...].astype(bf16)
    wb = wb_s[...]
    wadr = wb[:, 0:W]
    wadi = wb[:, W:2 * W]

    def dott(a, b):
        return jax.lax.dot_general(a, b, _DN_T, preferred_element_type=f32)

    s = dott(xr, wadr) + dott(xi, wadi)
    a_re = s[:, 0:M].reshape(G, H, M)
    a_im = s[:, M:2 * M].reshape(G, H, M)

    eyc = eyc_s[...][None]
    eys = eys_s[...][None]

    ks_re = jnp.sum(a_re * eyc + a_im * eys, axis=1, keepdims=True)
    ks_im = jnp.sum(a_im * eyc - a_re * eys, axis=1, keepdims=True)

    ksr = ks_re.astype(bf16)
    ksi = ks_im.astype(bf16)
    uc_s[:, 0:M] = (ksr * eyc - ksi * eys).reshape(G * H, M)
    uc_s[:, M:2 * M] = (ksr * eys + ksi * eyc).reshape(G * H, M)
    uc = uc_s[...]

    adj = jnp.dot(uc, wb, preferred_element_type=f32)
    adj_re = adj[:, 0:W]
    adj_im = adj[:, W:2 * W]
    mag = jnp.sqrt(adj_re * adj_re + adj_im * adj_im)
    out_ref[...] = mag.astype(out_ref.dtype)
    part = jnp.sum(mag, axis=0, keepdims=True)[None]

    @pl.when(i == 0)
    def _init_psum():
        psum_ref[...] = part

    @pl.when((i > 0) & (i < n - 1))
    def _acc_psum():
        psum_ref[...] += part

    @pl.when((i == n - 1) & (i > 0))
    def _finish_psum():
        total = psum_ref[...] + part
        psum_ref[...] = jnp.broadcast_to(
            jnp.sum(total, axis=-1, keepdims=True), total.shape)


def _forward(x_re, x_im, control):
    B, C, H, W = x_re.shape
    BC = B * C
    R = BC * H

    Nc, Nctrl, _ = control.shape
    Wtraj = jnp.asarray(_traj_matrix(Nctrl, 3))
    traj = jnp.einsum('jk,nkd->njd', Wtraj, control,
                      precision=jax.lax.Precision.HIGHEST).reshape(-1, 2)
    M = traj.shape[0]

    ax = _TWO_PI * traj[:, 0].astype(jnp.float32)
    ay = _TWO_PI * traj[:, 1].astype(jnp.float32)

    aux = jnp.concatenate(
        [jnp.broadcast_to(ax[:, None], (M, M)),
         jnp.broadcast_to(ay[None, :], (8, M))], axis=0)

    xr = x_re.reshape(R, W)
    xi = x_im.reshape(R, W)

    G = 8
    while BC % G != 0 or BC // G < 2:
        G //= 2
        if G == 1:
            break
    rows = G * H
    n_prog = R // rows
    grid = (n_prog,)

    kernel_fn = lambda *refs: _fused_ndft_kernel(G, H, M, W, *refs)

    mag, psum = pl.pallas_call(
        kernel_fn,
        out_shape=(jax.ShapeDtypeStruct((R, W), jnp.bfloat16),
                   jax.ShapeDtypeStruct((1, 1, W), jnp.float32)),
        grid=grid,
        in_specs=[
            pl.BlockSpec((rows, W), lambda i: (i, 0)),
            pl.BlockSpec((rows, W), lambda i: (i, 0)),
            pl.BlockSpec((M + 8, M), lambda i: (0, 0)),
        ],
        out_specs=(pl.BlockSpec((rows, W), lambda i: (i, 0)),
                   pl.BlockSpec((1, 1, W), lambda i: (0, 0, 0))),
        scratch_shapes=[pltpu.VMEM((2 * M, 2 * W), jnp.bfloat16),
                        pltpu.VMEM((H, M), jnp.bfloat16),
                        pltpu.VMEM((H, M), jnp.bfloat16),
                        pltpu.VMEM((rows, 2 * M), jnp.bfloat16)],
        compiler_params=pltpu.CompilerParams(
            dimension_semantics=("arbitrary",),
            vmem_limit_bytes=100 * 1024 * 1024),
    )(xr, xi, aux)

    mean = psum[0, 0, 0] / float(R * W)
    out = mag.astype(jnp.float32) * (1.0 / mean)
    return out.reshape(B, C, H, W)


_forward_jit = jax.jit(_forward)


def kernel(x_re, x_im, control):
    return _forward_jit(x_re, x_im, control)


# speedup vs baseline: 1.0003x; 1.0003x over previous
import numpy as np
import jax
import jax.numpy as jnp
from jax.experimental import pallas as pl
from jax.experimental.pallas import tpu as pltpu

_TWO_PI = float(2.0 * np.pi)
_HALF_PI = float(0.5 * np.pi)
_DN_T = (((1,), (1,)), ((), ()))


def _upsample2_matrix(L):
    Lout = 2 * L
    Wm = np.zeros((Lout, L), dtype=np.float32)
    if L == 1:
        Wm[:, 0] = 1.0
        return Wm
    j = np.arange(Lout, dtype=np.float32)
    pos = j * (L - 1) / (Lout - 1)
    i0 = np.clip(np.floor(pos).astype(np.int64), 0, L - 2)
    frac = (pos - i0).astype(np.float32)
    Wm[np.arange(Lout), i0] = 1.0 - frac
    Wm[np.arange(Lout), i0 + 1] = frac
    return Wm


def _traj_matrix(L, doublings):
    Wm = np.eye(L, dtype=np.float32)
    cur = L
    for _ in range(doublings):
        Wm = _upsample2_matrix(cur) @ Wm
        cur *= 2
    return Wm


def _fused_ndft_kernel(G, H, M, W,
                       xr_ref, xi_ref, aux_ref,
                       out_ref, psum_ref,
                       wb_s, eyc_s, eys_s, uc_s):
    f32 = jnp.float32
    bf16 = jnp.bfloat16
    i = pl.program_id(0)
    n = pl.num_programs(0)

    @pl.when(i == 0)
    def _build_tables():
        aux = aux_ref[...]
        ax = aux[0:M, 0:1]
        xp = (jax.lax.broadcasted_iota(jnp.int32, (M, W), 1)
              .astype(f32) - float(W // 2))
        ph = ax * xp
        cph = jnp.cos(ph)
        sph = jnp.sin(ph)
        wb_s[0:M, 0:W] = cph.astype(bf16)
        wb_s[M:2 * M, 0:W] = (-sph).astype(bf16)
        wb_s[0:M, W:2 * W] = sph.astype(bf16)
        wb_s[M:2 * M, W:2 * W] = cph.astype(bf16)
        ay = aux[M:M + 1, :]
        yp = (jax.lax.broadcasted_iota(jnp.int32, (H, M), 0)
              .astype(f32) - float(H // 2))
        ph_y = yp * ay
        eyc_s[...] = jnp.cos(ph_y).astype(bf16)
        eys_s[...] = jnp.sin(ph_y).astype(bf16)

    xr = xr_ref[...].astype(bf16)
    xi = xi_ref[...].astype(bf16)
    wb = wb_s[...]
    wadr = wb[:, 0:W]
    wadi = wb[:, W:2 * W]

    def dott(a, b):
        return jax.lax.dot_general(a, b, _DN_T, preferred_element_type=f32)

    excm = wadr[0:M, :]
    exsm = wadi[0:M, :]
    a_re = (dott(xr, excm) + dott(xi, exsm)).reshape(G, H, M)
    a_im = (dott(xi, excm) - dott(xr, exsm)).reshape(G, H, M)

    eyc = eyc_s[...][None]
    eys = eys_s[...][None]

    ks_re = jnp.sum(a_re * eyc + a_im * eys, axis=1, keepdims=True)
    ks_im = jnp.sum(a_im * eyc - a_re * eys, axis=1, keepdims=True)

    ksr = ks_re.astype(bf16)
    ksi = ks_im.astype(bf16)
    uc_s[:, 0:M] = (ksr * eyc - ksi * eys).reshape(G * H, M)
    uc_s[:, M:2 * M] = (ksr * eys + ksi * eyc).reshape(G * H, M)
    uc = uc_s[...]

    adj = jnp.dot(uc, wb, preferred_element_type=f32)
    adj_re = adj[:, 0:W]
    adj_im = adj[:, W:2 * W]
    mag = jnp.sqrt(adj_re * adj_re + adj_im * adj_im)
    out_ref[...] = mag.astype(out_ref.dtype)
    part = jnp.sum(mag, axis=0, keepdims=True)[None]

    @pl.when(i == 0)
    def _init_psum():
        psum_ref[...] = part

    @pl.when((i > 0) & (i < n - 1))
    def _acc_psum():
        psum_ref[...] += part

    @pl.when((i == n - 1) & (i > 0))
    def _finish_psum():
        total = psum_ref[...] + part
        psum_ref[...] = jnp.broadcast_to(
            jnp.sum(total, axis=-1, keepdims=True), total.shape)


def _forward(x_re, x_im, control):
    B, C, H, W = x_re.shape
    BC = B * C
    R = BC * H

    Nc, Nctrl, _ = control.shape
    Wtraj = jnp.asarray(_traj_matrix(Nctrl, 3))
    traj = jnp.einsum('jk,nkd->njd', Wtraj, control,
                      precision=jax.lax.Precision.HIGHEST).reshape(-1, 2)
    M = traj.shape[0]

    ax = _TWO_PI * traj[:, 0].astype(jnp.float32)
    ay = _TWO_PI * traj[:, 1].astype(jnp.float32)

    aux = jnp.concatenate(
        [jnp.broadcast_to(ax[:, None], (M, M)),
         jnp.broadcast_to(ay[None, :], (8, M))], axis=0)

    xr = x_re.reshape(R, W)
    xi = x_im.reshape(R, W)

    G = 8
    while BC % G != 0 or BC // G < 2:
        G //= 2
        if G == 1:
            break
    rows = G * H
    n_prog = R // rows
    grid = (n_prog,)

    kernel_fn = lambda *refs: _fused_ndft_kernel(G, H, M, W, *refs)

    mag, psum = pl.pallas_call(
        kernel_fn,
        out_shape=(jax.ShapeDtypeStruct((R, W), jnp.bfloat16),
                   jax.ShapeDtypeStruct((1, 1, W), jnp.float32)),
        grid=grid,
        in_specs=[
            pl.BlockSpec((rows, W), lambda i: (i, 0)),
            pl.BlockSpec((rows, W), lambda i: (i, 0)),
            pl.BlockSpec((M + 8, M), lambda i: (0, 0)),
        ],
        out_specs=(pl.BlockSpec((rows, W), lambda i: (i, 0)),
                   pl.BlockSpec((1, 1, W), lambda i: (0, 0, 0))),
        scratch_shapes=[pltpu.VMEM((2 * M, 2 * W), jnp.bfloat16),
                        pltpu.VMEM((H, M), jnp.bfloat16),
                        pltpu.VMEM((H, M), jnp.bfloat16),
                        pltpu.VMEM((rows, 2 * M), jnp.bfloat16)],
        compiler_params=pltpu.CompilerParams(
            dimension_semantics=("arbitrary",),
            vmem_limit_bytes=100 * 1024 * 1024),
    )(xr, xi, aux)

    mean = psum[0, 0, 0] / float(R * W)
    out = mag.astype(jnp.float32) * (1.0 / mean)
    return out.reshape(B, C, H, W)


_forward_jit = jax.jit(_forward)


def kernel(x_re, x_im, control):
    return _forward_jit(x_re, x_im, control)


# speedup vs baseline: 1.0027x; 1.0024x over previous
import numpy as np
import jax
import jax.numpy as jnp
from jax.experimental import pallas as pl
from jax.experimental.pallas import tpu as pltpu

_TWO_PI = float(2.0 * np.pi)
_HALF_PI = float(0.5 * np.pi)
_DN_T = (((1,), (1,)), ((), ()))


def _upsample2_matrix(L):
    Lout = 2 * L
    Wm = np.zeros((Lout, L), dtype=np.float32)
    if L == 1:
        Wm[:, 0] = 1.0
        return Wm
    j = np.arange(Lout, dtype=np.float32)
    pos = j * (L - 1) / (Lout - 1)
    i0 = np.clip(np.floor(pos).astype(np.int64), 0, L - 2)
    frac = (pos - i0).astype(np.float32)
    Wm[np.arange(Lout), i0] = 1.0 - frac
    Wm[np.arange(Lout), i0 + 1] = frac
    return Wm


def _traj_matrix(L, doublings):
    Wm = np.eye(L, dtype=np.float32)
    cur = L
    for _ in range(doublings):
        Wm = _upsample2_matrix(cur) @ Wm
        cur *= 2
    return Wm


def _fused_ndft_kernel(G, H, M, W,
                       xr_ref, xi_ref, aux_ref,
                       out_ref, psum_ref,
                       wb_s, eyc_s, eys_s, uc_s):
    f32 = jnp.float32
    bf16 = jnp.bfloat16
    i = pl.program_id(0)
    n = pl.num_programs(0)

    @pl.when(i == 0)
    def _build_tables():
        aux = aux_ref[...]
        ax = aux[0:M, 0:1]
        xp = (jax.lax.broadcasted_iota(jnp.int32, (M, W), 1)
              .astype(f32) - float(W // 2))
        ph = ax * xp
        cph = jnp.cos(ph)
        sph = jnp.sin(ph)
        wb_s[0:M, 0:W] = cph.astype(bf16)
        wb_s[M:2 * M, 0:W] = (-sph).astype(bf16)
        wb_s[0:M, W:2 * W] = sph.astype(bf16)
        wb_s[M:2 * M, W:2 * W] = cph.astype(bf16)
        ay = aux[M:M + 1, :]
        yp = (jax.lax.broadcasted_iota(jnp.int32, (H, M), 0)
              .astype(f32) - float(H // 2))
        ph_y = yp * ay
        eyc_s[...] = jnp.cos(ph_y).astype(bf16)
        eys_s[...] = jnp.sin(ph_y).astype(bf16)

    xr = xr_ref[...].astype(bf16)
    xi = xi_ref[...].astype(bf16)
    wb = wb_s[...]
    wadr = wb[:, 0:W]
    wadi = wb[:, W:2 * W]

    def dott(a, b):
        return jax.lax.dot_general(a, b, _DN_T, preferred_element_type=f32)

    excm = wadr[0:M, :]
    exsm = wadi[0:M, :]
    a_re = (dott(xr, excm) + dott(xi, exsm)).reshape(G, H, M)
    a_im = (dott(xi, excm) - dott(xr, exsm)).reshape(G, H, M)

    eyc = eyc_s[...][None]
    eys = eys_s[...][None]

    ks_re = jnp.sum(a_re * eyc + a_im * eys, axis=1, keepdims=True)
    ks_im = jnp.sum(a_im * eyc - a_re * eys, axis=1, keepdims=True)

    ksr = ks_re.astype(bf16)
    ksi = ks_im.astype(bf16)
    uc_s[:, 0:M] = (ksr * eyc - ksi * eys).reshape(G * H, M)
    uc_s[:, M:2 * M] = (ksr * eys + ksi * eyc).reshape(G * H, M)
    uc = uc_s[...]

    adj_re = jnp.dot(uc, wadr, preferred_element_type=f32)
    adj_im = jnp.dot(uc, wadi, preferred_element_type=f32)
    mag = jnp.sqrt(adj_re * adj_re + adj_im * adj_im)
    out_ref[...] = mag.astype(out_ref.dtype)
    part = jnp.sum(mag, axis=0, keepdims=True)[None]

    @pl.when(i == 0)
    def _init_psum():
        psum_ref[...] = part

    @pl.when((i > 0) & (i < n - 1))
    def _acc_psum():
        psum_ref[...] += part

    @pl.when((i == n - 1) & (i > 0))
    def _finish_psum():
        total = psum_ref[...] + part
        psum_ref[...] = jnp.broadcast_to(
            jnp.sum(total, axis=-1, keepdims=True), total.shape)


def _forward(x_re, x_im, control):
    B, C, H, W = x_re.shape
    BC = B * C
    R = BC * H

    Nc, Nctrl, _ = control.shape
    Wtraj = jnp.asarray(_traj_matrix(Nctrl, 3))
    traj = jnp.einsum('jk,nkd->njd', Wtraj, control,
                      precision=jax.lax.Precision.HIGHEST).reshape(-1, 2)
    M = traj.shape[0]

    ax = _TWO_PI * traj[:, 0].astype(jnp.float32)
    ay = _TWO_PI * traj[:, 1].astype(jnp.float32)

    aux = jnp.concatenate(
        [jnp.broadcast_to(ax[:, None], (M, M)),
         jnp.broadcast_to(ay[None, :], (8, M))], axis=0)

    xr = x_re.reshape(R, W)
    xi = x_im.reshape(R, W)

    G = 8
    while BC % G != 0 or BC // G < 2:
        G //= 2
        if G == 1:
            break
    rows = G * H
    n_prog = R // rows
    grid = (n_prog,)

    kernel_fn = lambda *refs: _fused_ndft_kernel(G, H, M, W, *refs)

    mag, psum = pl.pallas_call(
        kernel_fn,
        out_shape=(jax.ShapeDtypeStruct((R, W), jnp.bfloat16),
                   jax.ShapeDtypeStruct((1, 1, W), jnp.float32)),
        grid=grid,
        in_specs=[
            pl.BlockSpec((rows, W), lambda i: (i, 0)),
            pl.BlockSpec((rows, W), lambda i: (i, 0)),
            pl.BlockSpec((M + 8, M), lambda i: (0, 0)),
        ],
        out_specs=(pl.BlockSpec((rows, W), lambda i: (i, 0)),
                   pl.BlockSpec((1, 1, W), lambda i: (0, 0, 0))),
        scratch_shapes=[pltpu.VMEM((2 * M, 2 * W), jnp.bfloat16),
                        pltpu.VMEM((H, M), jnp.bfloat16),
                        pltpu.VMEM((H, M), jnp.bfloat16),
                        pltpu.VMEM((rows, 2 * M), jnp.bfloat16)],
        compiler_params=pltpu.CompilerParams(
            dimension_semantics=("arbitrary",),
            vmem_limit_bytes=100 * 1024 * 1024),
    )(xr, xi, aux)

    mean = psum[0, 0, 0] / float(R * W)
    out = mag.astype(jnp.float32) * (1.0 / mean)
    return out.reshape(B, C, H, W)


_forward_jit = jax.jit(_forward)


def kernel(x_re, x_im, control):
    return _forward_jit(x_re, x_im, control)


# speedup vs baseline: 1.0956x; 1.0927x over previous
import numpy as np
import jax
import jax.numpy as jnp
from jax.experimental import pallas as pl
from jax.experimental.pallas import tpu as pltpu

_TWO_PI = float(2.0 * np.pi)
_HALF_PI = float(0.5 * np.pi)
_DN_T = (((1,), (1,)), ((), ()))


def _upsample2_matrix(L):
    Lout = 2 * L
    Wm = np.zeros((Lout, L), dtype=np.float32)
    if L == 1:
        Wm[:, 0] = 1.0
        return Wm
    j = np.arange(Lout, dtype=np.float32)
    pos = j * (L - 1) / (Lout - 1)
    i0 = np.clip(np.floor(pos).astype(np.int64), 0, L - 2)
    frac = (pos - i0).astype(np.float32)
    Wm[np.arange(Lout), i0] = 1.0 - frac
    Wm[np.arange(Lout), i0 + 1] = frac
    return Wm


def _traj_matrix(L, doublings):
    Wm = np.eye(L, dtype=np.float32)
    cur = L
    for _ in range(doublings):
        Wm = _upsample2_matrix(cur) @ Wm
        cur *= 2
    return Wm


def _fused_ndft_kernel(G, H, M, W, R,
                       xr_ref, xi_ref, aux_ref,
                       out_ref,
                       wb_s, eyc_s, eys_s, uc_s, mag_s, psum_s):
    f32 = jnp.float32
    bf16 = jnp.bfloat16
    i = pl.program_id(0)
    n = pl.num_programs(0) // 2

    @pl.when(i == 0)
    def _build_tables():
        aux = aux_ref[...]
        ax = aux[0:M, 0:1]
        xp = (jax.lax.broadcasted_iota(jnp.int32, (M, W), 1)
              .astype(f32) - float(W // 2))
        ph = ax * xp
        cph = jnp.cos(ph)
        sph = jnp.sin(ph)
        wb_s[0:M, 0:W] = cph.astype(bf16)
        wb_s[M:2 * M, 0:W] = (-sph).astype(bf16)
        wb_s[0:M, W:2 * W] = sph.astype(bf16)
        wb_s[M:2 * M, W:2 * W] = cph.astype(bf16)
        ay = aux[M:M + 1, :]
        yp = (jax.lax.broadcasted_iota(jnp.int32, (H, M), 0)
              .astype(f32) - float(H // 2))
        ph_y = yp * ay
        eyc_s[...] = jnp.cos(ph_y).astype(bf16)
        eys_s[...] = jnp.sin(ph_y).astype(bf16)

    rows = G * H

    @pl.when(i < n)
    def _compute_phase():
        xr = xr_ref[...].astype(bf16)
        xi = xi_ref[...].astype(bf16)
        wb = wb_s[...]
        wadr = wb[:, 0:W]
        wadi = wb[:, W:2 * W]

        def dott(a, b):
            return jax.lax.dot_general(a, b, _DN_T,
                                       preferred_element_type=f32)

        excm = wadr[0:M, :]
        exsm = wadi[0:M, :]
        a_re = (dott(xr, excm) + dott(xi, exsm)).reshape(G, H, M)
        a_im = (dott(xi, excm) - dott(xr, exsm)).reshape(G, H, M)

        eyc = eyc_s[...][None]
        eys = eys_s[...][None]

        ks_re = jnp.sum(a_re * eyc + a_im * eys, axis=1, keepdims=True)
        ks_im = jnp.sum(a_im * eyc - a_re * eys, axis=1, keepdims=True)

        ksr = ks_re.astype(bf16)
        ksi = ks_im.astype(bf16)
        uc_s[:, 0:M] = (ksr * eyc - ksi * eys).reshape(G * H, M)
        uc_s[:, M:2 * M] = (ksr * eys + ksi * eyc).reshape(G * H, M)
        uc = uc_s[...]

        adj_re = jnp.dot(uc, wadr, preferred_element_type=f32)
        adj_im = jnp.dot(uc, wadi, preferred_element_type=f32)
        mag = jnp.sqrt(adj_re * adj_re + adj_im * adj_im)
        mag_s[pl.ds(i * rows, rows), :] = mag.astype(bf16)
        part = jnp.sum(mag, axis=0, keepdims=True)

        @pl.when(i == 0)
        def _init_psum():
            psum_s[0:1, :] = part

        @pl.when(i > 0)
        def _acc_psum():
            psum_s[0:1, :] += part

    @pl.when(i >= n)
    def _scale_phase():
        total = jnp.sum(psum_s[0:1, :])
        scale = float(R * W) / total
        j = i - n
        out_ref[...] = mag_s[pl.ds(j * rows, rows), :].astype(f32) * scale


def _forward(x_re, x_im, control):
    B, C, H, W = x_re.shape
    BC = B * C
    R = BC * H

    Nc, Nctrl, _ = control.shape
    Wtraj = jnp.asarray(_traj_matrix(Nctrl, 3))
    traj = jnp.einsum('jk,nkd->njd', Wtraj, control,
                      precision=jax.lax.Precision.HIGHEST).reshape(-1, 2)
    M = traj.shape[0]

    ax = _TWO_PI * traj[:, 0].astype(jnp.float32)
    ay = _TWO_PI * traj[:, 1].astype(jnp.float32)

    aux = jnp.concatenate(
        [jnp.broadcast_to(ax[:, None], (M, M)),
         jnp.broadcast_to(ay[None, :], (8, M))], axis=0)

    xr = x_re.reshape(R, W)
    xi = x_im.reshape(R, W)

    G = 8
    while BC % G != 0 or BC // G < 2:
        G //= 2
        if G == 1:
            break
    rows = G * H
    n_prog = R // rows
    grid = (2 * n_prog,)

    kernel_fn = lambda *refs: _fused_ndft_kernel(G, H, M, W, R, *refs)

    out = pl.pallas_call(
        kernel_fn,
        out_shape=jax.ShapeDtypeStruct((R, W), jnp.float32),
        grid=grid,
        in_specs=[
            pl.BlockSpec((rows, W),
                         lambda i: (jnp.minimum(i, n_prog - 1), 0)),
            pl.BlockSpec((rows, W),
                         lambda i: (jnp.minimum(i, n_prog - 1), 0)),
            pl.BlockSpec((M + 8, M), lambda i: (0, 0)),
        ],
        out_specs=pl.BlockSpec((rows, W),
                               lambda i: (jnp.maximum(i - n_prog, 0), 0)),
        scratch_shapes=[pltpu.VMEM((2 * M, 2 * W), jnp.bfloat16),
                        pltpu.VMEM((H, M), jnp.bfloat16),
                        pltpu.VMEM((H, M), jnp.bfloat16),
                        pltpu.VMEM((rows, 2 * M), jnp.bfloat16),
                        pltpu.VMEM((R, W), jnp.bfloat16),
                        pltpu.VMEM((8, W), jnp.float32)],
        compiler_params=pltpu.CompilerParams(
            dimension_semantics=("arbitrary",),
            vmem_limit_bytes=100 * 1024 * 1024),
    )(xr, xi, aux)

    return out.reshape(B, C, H, W)


_forward_jit = jax.jit(_forward)


def kernel(x_re, x_im, control):
    return _forward_jit(x_re, x_im, control)


# speedup vs baseline: 1.1218x; 1.0239x over previous
import numpy as np
import jax
import jax.numpy as jnp
from jax.experimental import pallas as pl
from jax.experimental.pallas import tpu as pltpu

_TWO_PI = float(2.0 * np.pi)
_HALF_PI = float(0.5 * np.pi)
_DN_T = (((1,), (1,)), ((), ()))


def _upsample2_matrix(L):
    Lout = 2 * L
    Wm = np.zeros((Lout, L), dtype=np.float32)
    if L == 1:
        Wm[:, 0] = 1.0
        return Wm
    j = np.arange(Lout, dtype=np.float32)
    pos = j * (L - 1) / (Lout - 1)
    i0 = np.clip(np.floor(pos).astype(np.int64), 0, L - 2)
    frac = (pos - i0).astype(np.float32)
    Wm[np.arange(Lout), i0] = 1.0 - frac
    Wm[np.arange(Lout), i0 + 1] = frac
    return Wm


def _traj_matrix(L, doublings):
    Wm = np.eye(L, dtype=np.float32)
    cur = L
    for _ in range(doublings):
        Wm = _upsample2_matrix(cur) @ Wm
        cur *= 2
    return Wm


def _fused_ndft_kernel(G, H, M, W, R,
                       xr_ref, xi_ref, aux_ref,
                       out_ref,
                       wb_s, eyc_s, eys_s, uc_s, mag_s, psum_s):
    f32 = jnp.float32
    bf16 = jnp.bfloat16
    i = pl.program_id(0)
    n = pl.num_programs(0) // 2

    @pl.when(i == 0)
    def _build_tables():
        aux = aux_ref[...]
        ax = aux[0:M, 0:1]
        xp = (jax.lax.broadcasted_iota(jnp.int32, (M, W), 1)
              .astype(f32) - float(W // 2))
        ph = ax * xp
        cph = jnp.cos(ph)
        sph = jnp.sin(ph)
        wb_s[0:M, 0:W] = cph.astype(bf16)
        wb_s[M:2 * M, 0:W] = (-sph).astype(bf16)
        wb_s[0:M, W:2 * W] = sph.astype(bf16)
        wb_s[M:2 * M, W:2 * W] = cph.astype(bf16)
        ay = aux[M:M + 1, :]
        yp = (jax.lax.broadcasted_iota(jnp.int32, (H, M), 0)
              .astype(f32) - float(H // 2))
        ph_y = yp * ay
        eyc_s[...] = jnp.cos(ph_y).astype(bf16)
        eys_s[...] = jnp.sin(ph_y).astype(bf16)

    rows = G * H

    @pl.when(i < n)
    def _compute_phase():
        xr = xr_ref[...].astype(bf16)
        xi = xi_ref[...].astype(bf16)
        wb = wb_s[...]
        wadr = wb[:, 0:W]
        wadi = wb[:, W:2 * W]

        def dott(a, b):
            return jax.lax.dot_general(a, b, _DN_T,
                                       preferred_element_type=f32)

        excm = wadr[0:M, :]
        exsm = wadi[0:M, :]
        a_re = (dott(xr, excm) + dott(xi, exsm)).reshape(G, H, M)
        a_im = (dott(xi, excm) - dott(xr, exsm)).reshape(G, H, M)

        eyc = eyc_s[...][None]
        eys = eys_s[...][None]

        ks_re = jnp.sum(a_re * eyc + a_im * eys, axis=1, keepdims=True)
        ks_im = jnp.sum(a_im * eyc - a_re * eys, axis=1, keepdims=True)

        ksr = ks_re.astype(bf16)
        ksi = ks_im.astype(bf16)
        uc_s[:, 0:M] = (ksr * eyc - ksi * eys).reshape(G * H, M)
        uc_s[:, M:2 * M] = (ksr * eys + ksi * eyc).reshape(G * H, M)
        uc = uc_s[...]

        adj_re = jnp.dot(uc, wadr, preferred_element_type=f32)
        adj_im = jnp.dot(uc, wadi, preferred_element_type=f32)
        mag = jnp.sqrt(adj_re * adj_re + adj_im * adj_im)
        mag_s[pl.ds(i * rows, rows), :] = mag.astype(bf16)
        part = jnp.sum(mag, axis=0, keepdims=True)

        @pl.when(i == 0)
        def _init_psum():
            psum_s[0:1, :] = part

        @pl.when(i > 0)
        def _acc_psum():
            psum_s[0:1, :] += part

    @pl.when(i >= n)
    def _scale_phase():
        total = jnp.sum(psum_s[0:1, :])
        scale = float(R * W) / total
        j = i - n
        out_ref[...] = mag_s[pl.ds(j * rows, rows), :].astype(f32) * scale


def _forward(x_re, x_im, control):
    B, C, H, W = x_re.shape
    BC = B * C
    R = BC * H

    Nc, Nctrl, _ = control.shape
    Wtraj = jnp.asarray(_traj_matrix(Nctrl, 3))
    traj = jnp.einsum('jk,nkd->njd', Wtraj, control,
                      precision=jax.lax.Precision.HIGHEST).reshape(-1, 2)
    M = traj.shape[0]

    ax = _TWO_PI * traj[:, 0].astype(jnp.float32)
    ay = _TWO_PI * traj[:, 1].astype(jnp.float32)

    aux = jnp.concatenate(
        [jnp.broadcast_to(ax[:, None], (M, M)),
         jnp.broadcast_to(ay[None, :], (8, M))], axis=0)

    xr = x_re.reshape(R, W)
    xi = x_im.reshape(R, W)

    G = 16
    while BC % G != 0 or BC // G < 2:
        G //= 2
        if G == 1:
            break
    rows = G * H
    n_prog = R // rows
    grid = (2 * n_prog,)

    kernel_fn = lambda *refs: _fused_ndft_kernel(G, H, M, W, R, *refs)

    out = pl.pallas_call(
        kernel_fn,
        out_shape=jax.ShapeDtypeStruct((R, W), jnp.float32),
        grid=grid,
        in_specs=[
            pl.BlockSpec((rows, W),
                         lambda i: (jnp.minimum(i, n_prog - 1), 0)),
            pl.BlockSpec((rows, W),
                         lambda i: (jnp.minimum(i, n_prog - 1), 0)),
            pl.BlockSpec((M + 8, M), lambda i: (0, 0)),
        ],
        out_specs=pl.BlockSpec((rows, W),
                               lambda i: (jnp.maximum(i - n_prog, 0), 0)),
        scratch_shapes=[pltpu.VMEM((2 * M, 2 * W), jnp.bfloat16),
                        pltpu.VMEM((H, M), jnp.bfloat16),
                        pltpu.VMEM((H, M), jnp.bfloat16),
                        pltpu.VMEM((rows, 2 * M), jnp.bfloat16),
                        pltpu.VMEM((R, W), jnp.bfloat16),
                        pltpu.VMEM((8, W), jnp.float32)],
        compiler_params=pltpu.CompilerParams(
            dimension_semantics=("arbitrary",),
            vmem_limit_bytes=100 * 1024 * 1024),
    )(xr, xi, aux)

    return out.reshape(B, C, H, W)


_forward_jit = jax.jit(_forward)


def kernel(x_re, x_im, control):
    return _forward_jit(x_re, x_im, control)


# speedup vs baseline: 1.1219x; 1.0001x over previous
import numpy as np
import jax
import jax.numpy as jnp
from jax.experimental import pallas as pl
from jax.experimental.pallas import tpu as pltpu

_TWO_PI = float(2.0 * np.pi)
_DN_T = (((1,), (1,)), ((), ()))


def _upsample2_matrix(L):
    Lout = 2 * L
    Wm = np.zeros((Lout, L), dtype=np.float32)
    if L == 1:
        Wm[:, 0] = 1.0
        return Wm
    j = np.arange(Lout, dtype=np.float32)
    pos = j * (L - 1) / (Lout - 1)
    i0 = np.clip(np.floor(pos).astype(np.int64), 0, L - 2)
    frac = (pos - i0).astype(np.float32)
    Wm[np.arange(Lout), i0] = 1.0 - frac
    Wm[np.arange(Lout), i0 + 1] = frac
    return Wm


def _traj_matrix(L, doublings):
    Wm = np.eye(L, dtype=np.float32)
    cur = L
    for _ in range(doublings):
        Wm = _upsample2_matrix(cur) @ Wm
        cur *= 2
    return Wm


def _fused_ndft_kernel(G, H, M, W, R,
                       xr_ref, xi_ref, aux_ref,
                       out_ref,
                       wb_s, eyc_s, eys_s, uc_s, mag_s, psum_s):
    f32 = jnp.float32
    bf16 = jnp.bfloat16
    i = pl.program_id(0)
    n = pl.num_programs(0) // 2

    @pl.when(i == 0)
    def _build_tables():
        aux = aux_ref[...]
        ax = aux[0:M, 0:1]
        xp = (jax.lax.broadcasted_iota(jnp.int32, (M, W), 1)
              .astype(f32) - float(W // 2))
        ph = ax * xp
        cph = jnp.cos(ph)
        sph = jnp.sin(ph)
        wb_s[0:M, 0:W] = cph.astype(bf16)
        wb_s[M:2 * M, 0:W] = (-sph).astype(bf16)
        wb_s[0:M, W:2 * W] = sph.astype(bf16)
        wb_s[M:2 * M, W:2 * W] = cph.astype(bf16)
        ay = aux[M:M + 1, :]
        yp = (jax.lax.broadcasted_iota(jnp.int32, (H, M), 0)
              .astype(f32) - float(H // 2))
        ph_y = yp * ay
        eyc_s[...] = jnp.cos(ph_y).astype(bf16)
        eys_s[...] = jnp.sin(ph_y).astype(bf16)

    rows = G * H

    @pl.when(i < n)
    def _compute_phase():
        xr = xr_ref[...].astype(bf16)
        xi = xi_ref[...].astype(bf16)
        wb = wb_s[...]
        wadr = wb[:, 0:W]
        wadi = wb[:, W:2 * W]

        def dott(a, b):
            return jax.lax.dot_general(a, b, _DN_T,
                                       preferred_element_type=f32)

        excm = wadr[0:M, :]
        exsm = wadi[0:M, :]
        a_re = (dott(xr, excm) + dott(xi, exsm)).reshape(G, H, M)
        a_im = (dott(xi, excm) - dott(xr, exsm)).reshape(G, H, M)

        eyc = eyc_s[...][None]
        eys = eys_s[...][None]

        ks_re = jnp.sum(a_re * eyc + a_im * eys, axis=1, keepdims=True)
        ks_im = jnp.sum(a_im * eyc - a_re * eys, axis=1, keepdims=True)

        ksr = ks_re.astype(bf16)
        ksi = ks_im.astype(bf16)
        uc_s[:, 0:M] = (ksr * eyc - ksi * eys).reshape(G * H, M)
        uc_s[:, M:2 * M] = (ksr * eys + ksi * eyc).reshape(G * H, M)
        uc = uc_s[...]

        adj_re = jnp.dot(uc, wadr, preferred_element_type=f32)
        adj_im = jnp.dot(uc, wadi, preferred_element_type=f32)
        mag = jnp.sqrt(adj_re * adj_re + adj_im * adj_im)
        mag_s[pl.ds(i * rows, rows), :] = mag.astype(bf16)
        part = jnp.sum(mag, axis=0, keepdims=True)

        @pl.when(i == 0)
        def _init_psum():
            psum_s[0:1, :] = part

        @pl.when(i > 0)
        def _acc_psum():
            psum_s[0:1, :] += part

    @pl.when(i >= n)
    def _scale_phase():
        total = jnp.sum(psum_s[0:1, :])
        scale = float(R * W) / total
        j = i - n
        out_ref[...] = mag_s[pl.ds(j * rows, rows), :].astype(f32) * scale


def _forward(x_re, x_im, control):
    B, C, H, W = x_re.shape
    BC = B * C
    R = BC * H

    Nc, Nctrl, _ = control.shape
    Wtraj = jnp.asarray(_traj_matrix(Nctrl, 3))
    traj = jnp.einsum('jk,nkd->njd', Wtraj, control,
                      precision=jax.lax.Precision.HIGHEST).reshape(-1, 2)
    M = traj.shape[0]

    ax = _TWO_PI * traj[:, 0].astype(jnp.float32)
    ay = _TWO_PI * traj[:, 1].astype(jnp.float32)

    aux = jnp.concatenate(
        [jnp.broadcast_to(ax[:, None], (M, M)),
         jnp.broadcast_to(ay[None, :], (8, M))], axis=0)

    xr = x_re.reshape(R, W)
    xi = x_im.reshape(R, W)

    G = 16
    while BC % G != 0 or BC // G < 2:
        G //= 2
        if G == 1:
            break
    rows = G * H
    n_prog = R // rows
    grid = (2 * n_prog,)

    kernel_fn = lambda *refs: _fused_ndft_kernel(G, H, M, W, R, *refs)

    out = pl.pallas_call(
        kernel_fn,
        out_shape=jax.ShapeDtypeStruct((R, W), jnp.float32),
        grid=grid,
        in_specs=[
            pl.BlockSpec((rows, W),
                         lambda i: (jnp.minimum(i, n_prog - 1), 0)),
            pl.BlockSpec((rows, W),
                         lambda i: (jnp.minimum(i, n_prog - 1), 0)),
            pl.BlockSpec((M + 8, M), lambda i: (0, 0)),
        ],
        out_specs=pl.BlockSpec((rows, W),
                               lambda i: (jnp.maximum(i - n_prog, 0), 0)),
        scratch_shapes=[pltpu.VMEM((2 * M, 2 * W), jnp.bfloat16),
                        pltpu.VMEM((H, M), jnp.bfloat16),
                        pltpu.VMEM((H, M), jnp.bfloat16),
                        pltpu.VMEM((rows, 2 * M), jnp.bfloat16),
                        pltpu.VMEM((R, W), jnp.bfloat16),
                        pltpu.VMEM((8, W), jnp.float32)],
        compiler_params=pltpu.CompilerParams(
            dimension_semantics=("arbitrary",),
            vmem_limit_bytes=100 * 1024 * 1024),
    )(xr, xi, aux)

    return out.reshape(B, C, H, W)


_forward_jit = jax.jit(_forward)


def kernel(x_re, x_im, control):
    return _forward_jit(x_re, x_im, control)
